# baseline (speedup 1.0000x reference)
import functools

import jax
import jax.numpy as jnp
from jax import lax
from jax.experimental import pallas as pl
from jax.experimental.pallas import tpu as pltpu
from jax.experimental.pallas import tpu_sc as plsc

_NUM_OBJS = 1000
_NUM_RELS = 64
_BATCH = 16384
_V = _NUM_OBJS * _NUM_OBJS
_L = 16
_BLK = 128
_NBLK = (_V + _BLK - 1) // _BLK
_TAILB = _NBLK - 1
_TAIL0 = _TAILB * _BLK
_RING = 6
_CAP = 16
_CHUNK = 2048


@functools.lru_cache(maxsize=None)
def _build(num_cores: int, num_subcores: int):
    nw = num_cores * num_subcores
    bpb = (_NBLK + nw - 1) // nw
    nvis = (bpb + _RING - 1) // _RING * _RING
    mesh = plsc.VectorSubcoreMesh(
        core_axis_name="c", subcore_axis_name="s",
        num_cores=num_cores, num_subcores=num_subcores)

    slot_types = [pltpu.VMEM((_NUM_RELS, _BLK), jnp.float32)
                  for _ in range(_RING)]
    lsem_types = [pltpu.SemaphoreType.DMA for _ in range(_RING)]
    esem_types = [pltpu.SemaphoreType.DMA for _ in range(_RING)]

    @functools.partial(
        pl.kernel,
        out_type=(jax.ShapeDtypeStruct((_BATCH, _NUM_RELS), jnp.float32),
                  jax.ShapeDtypeStruct((nw, _NUM_RELS), jnp.float32)),
        mesh=mesh,
        scratch_types=[
            pltpu.VMEM((_BATCH + _L,), jnp.int32),
            pltpu.VMEM((_BATCH + _L,), jnp.int32),
            pltpu.VMEM((_BATCH + _L,), jnp.int32),
            pltpu.VMEM((8192 + _L,), jnp.int32),
            pltpu.VMEM((_NUM_RELS // 2, _BLK), jnp.float32),
            pltpu.VMEM((_NUM_RELS,), jnp.int32),
            pltpu.VMEM((_NUM_RELS, _BLK), jnp.float32),
            *slot_types,
            *lsem_types,
            *esem_types,
            pltpu.SemaphoreType.DMA,
        ],
    )
    def k(sl0_hbm, sl1_hbm, pos_hbm, starts_hbm, table_hbm, tail_hbm,
          out_hbm, scrap_hbm, sl0_v, sl1_v, pos_v, starts_v, tail_v, dr_v,
          spare_v,
          *rest):
        slots = rest[:_RING]
        lsems = rest[_RING:2 * _RING]
        esems = rest[2 * _RING:3 * _RING]
        ovf_sem = rest[3 * _RING]
        wid = lax.axis_index("s") * num_cores + lax.axis_index("c")
        wbase = wid * bpb

        def issue_load(local_i, slot, sem):
            b = jnp.minimum(wbase + local_i, _TAILB - 1)
            off = pl.multiple_of(b * _BLK, _BLK)
            pltpu.async_copy(table_hbm.at[:, pl.ds(off, _BLK)], slot, sem)

        for s in range(_RING // 2):
            issue_load(s, slots[s], lsems[s])
        pltpu.sync_copy(tail_hbm, tail_v)
        for cs in range(4):
            slc = pl.ds(cs * 2048, 2048)
            pltpu.sync_copy(starts_hbm.at[slc], starts_v.at[slc])
        for cb in range(_BATCH // _CHUNK):
            sl = pl.ds(cb * _CHUNK, _CHUNK)
            pltpu.sync_copy(sl0_hbm.at[sl], sl0_v.at[sl])
            pltpu.sync_copy(sl1_hbm.at[sl], sl1_v.at[sl])
            pltpu.sync_copy(pos_hbm.at[sl], pos_v.at[sl])

        lane0 = lax.iota(jnp.int32, _L)

        def read_at(ref, q):
            qa = (q >> 3) << 3
            v = ref[pl.ds(qa, _L)]
            c = q & 7
            r = v[0]
            for jj in range(1, 8):
                r = jnp.where(c == jj, v[jj], r)
            return r

        def fused_at(q):
            return read_at(sl0_v, q) * _NUM_OBJS + read_at(sl1_v, q)

        def out_hit(p, r_m, block_b, resident_ref, sem):
            @pl.when(block_b == _TAILB)
            def _():
                pltpu.async_copy(
                    tail_v.at[r_m >> 1, pl.ds((r_m & 1) * _NUM_RELS,
                                              _NUM_RELS)],
                    out_hbm.at[p], sem)
            @pl.when(block_b != _TAILB)
            def _():
                pltpu.async_copy(resident_ref.at[:, r_m], out_hbm.at[p], sem)

        def active(local_i):
            return ((local_i >= 0) & (local_i < bpb)
                    & (wbase + local_i < _NBLK))

        def wave(t, _):
            for s in range(_RING):
                i = t * _RING + s
                j = (s + _RING // 2) % _RING
                @pl.when(active(i - _RING // 2))
                def _():
                    pltpu.make_async_copy(
                        table_hbm.at[pl.ds(0, 8), pl.ds(0, _BLK)],
                        slots[j].at[pl.ds(0, 8), :], esems[j]).wait()
                @pl.when(i + _RING // 2 < nvis)
                def _():
                    issue_load(i + _RING // 2, slots[j], lsems[j])
                @pl.when(i < nvis)
                def _():
                    pltpu.make_async_copy(
                        table_hbm.at[:, pl.ds(0, _BLK)],
                        slots[s], lsems[s]).wait()
                b = wbase + i
                is_act = active(i)

                q0 = read_at(starts_v, b)
                q1 = read_at(starts_v, b + 1)
                nh = q1 - q0
                cnt = jnp.minimum(nh, _CAP)

                @pl.when(is_act & (nh > 0) & (b != _TAILB))
                def _():
                    boff = pl.multiple_of(
                        jnp.minimum(b, _TAILB - 1) * _BLK, _BLK)
                    pltpu.sync_copy(table_hbm.at[:, pl.ds(boff, _BLK)],
                                    spare_v)

                def visit_hit(e, _):
                    q = jnp.minimum(q0 + e, _BATCH - 1)
                    r = fused_at(q)
                    out_hit(read_at(pos_v, q), r & (_BLK - 1), b,
                            spare_v, ovf_sem)
                    pltpu.make_async_copy(
                        sl0_hbm.at[pl.ds(0, _NUM_RELS)], dr_v,
                        ovf_sem).wait()
                    return 0
                lax.fori_loop(0, jnp.where(is_act, cnt, 0), visit_hit, 0)

                def pad(e, _):
                    pltpu.async_copy(slots[s].at[:, 0],
                                     scrap_hbm.at[wid], esems[s])
                    return 0
                lax.fori_loop(jnp.where(is_act, 0, _CAP), _CAP, pad, 0)

                def ovf(e, _):
                    q = jnp.minimum(q0 + _CAP + e, _BATCH - 1)
                    r = fused_at(q)
                    out_hit(read_at(pos_v, q), r & (_BLK - 1), b,
                            spare_v, ovf_sem)
                    pltpu.make_async_copy(
                        sl0_hbm.at[pl.ds(0, _NUM_RELS)], dr_v,
                        ovf_sem).wait()
                    return 0
                lax.fori_loop(0, jnp.where(is_act, nh - cnt, 0), ovf, 0)
            return 0

        lax.fori_loop(0, nvis // _RING, wave, 0)
        for s in range(_RING):
            i_last = nvis - _RING + s
            @pl.when(active(i_last) & (i_last + _RING // 2 >= nvis))
            def _():
                pltpu.make_async_copy(
                    table_hbm.at[pl.ds(0, 8), pl.ds(0, _BLK)],
                    slots[s].at[pl.ds(0, 8), :], esems[s]).wait()

    return k


def kernel(labels, obj_baseline):
    info = plsc.get_sparse_core_info()
    nw = info.num_cores * info.num_subcores
    bpb = (_NBLK + nw - 1) // nw
    k = _build(info.num_cores, info.num_subcores)
    l0 = labels[:, 0]
    l1 = labels[:, 1]
    fused = l0 * _NUM_OBJS + l1
    order = jnp.argsort(fused).astype(jnp.int32)
    sl0 = l0[order]
    sl1 = l1[order]
    sfused = fused[order]
    bounds = jnp.minimum(jnp.arange(8192, dtype=jnp.int32), _NBLK) * _BLK
    starts = jnp.searchsorted(sfused, bounds, side="left").astype(jnp.int32)
    tail = obj_baseline[_TAIL0:].reshape(_NUM_RELS // 2, _BLK)
    out, _scrap = k(sl0, sl1, order, starts, obj_baseline.T, tail)
    return out


# speedup vs baseline: 100.2812x; 100.2812x over previous
import functools

import jax
import jax.numpy as jnp
from jax import lax
from jax.experimental import pallas as pl
from jax.experimental.pallas import tpu as pltpu
from jax.experimental.pallas import tpu_sc as plsc

_NUM_OBJS = 1000
_NUM_RELS = 64
_BATCH = 16384
_L = 16


@functools.lru_cache(maxsize=None)
def _build(num_cores: int, num_subcores: int):
    nw = num_cores * num_subcores
    bpw = _BATCH // nw
    mesh = plsc.VectorSubcoreMesh(
        core_axis_name="c", subcore_axis_name="s",
        num_cores=num_cores, num_subcores=num_subcores)

    @functools.partial(
        pl.kernel,
        out_type=jax.ShapeDtypeStruct((_BATCH, _NUM_RELS), jnp.float32),
        mesh=mesh,
        scratch_types=[
            pltpu.VMEM((2 * bpw,), jnp.int32),
            pltpu.VMEM((bpw, _NUM_RELS), jnp.float32),
            pltpu.SemaphoreType.DMA,
        ],
        compiler_params=pltpu.CompilerParams(use_tc_tiling_on_sc=False),
    )
    def k(labels_hbm, table_hbm, out_hbm, lab_v, rows_v, sem):
        wid = lax.axis_index("s") * num_cores + lax.axis_index("c")
        base = wid * bpw
        pltpu.sync_copy(labels_hbm.at[pl.ds(base * 2, 2 * bpw)], lab_v)
        lane = lax.iota(jnp.int32, _L)
        half = lane < 8
        even2 = (lane & 7) * 2
        odd2 = even2 + 1
        copies = []
        for j in range(bpw // _L):
            a = lab_v[pl.ds(2 * _L * j, _L)]
            b = lab_v[pl.ds(2 * _L * j + _L, _L)]
            l0 = jnp.where(half,
                           a.at[even2].get(mode="promise_in_bounds"),
                           b.at[even2].get(mode="promise_in_bounds"))
            l1 = jnp.where(half,
                           a.at[odd2].get(mode="promise_in_bounds"),
                           b.at[odd2].get(mode="promise_in_bounds"))
            fused = l0 * _NUM_OBJS + l1
            copies.append(pltpu.async_copy(
                table_hbm.at[fused],
                rows_v.at[pl.ds(j * _L, _L)],
                sem))
        for cp in copies:
            cp.wait()
        pltpu.sync_copy(rows_v, out_hbm.at[pl.ds(base, bpw)])

    return k


def kernel(labels, obj_baseline):
    info = plsc.get_sparse_core_info()
    k = _build(info.num_cores, info.num_subcores)
    return k(labels.reshape(-1), obj_baseline)
